# Initial kernel scaffold; baseline (speedup 1.0000x reference)
import jax
import jax.numpy as jnp
from jax.experimental import pallas as pl


def kernel(x, edge_index, W0, W1):
    raise NotImplementedError("write your pallas kernel here")


# speedup vs baseline: 28.7495x; 28.7495x over previous
import functools

import jax
import jax.numpy as jnp
from jax import lax
from jax.experimental import pallas as pl
from jax.experimental.pallas import tpu as pltpu
from jax.experimental.pallas import tpu_sc as plsc

NC = 2
NS = 16
L = 16
CH = 128


def _round_up(a, b):
    return (a + b - 1) // b * b


def _make_deg_kernel(n_pad, nch):
    mesh = plsc.VectorSubcoreMesh(core_axis_name="c", subcore_axis_name="s")
    rpt = n_pad // NS

    @functools.partial(
        pl.kernel,
        out_type=jax.ShapeDtypeStruct((NC, n_pad), jnp.float32),
        mesh=mesh,
        compiler_params=pltpu.CompilerParams(use_tc_tiling_on_sc=False),
        scratch_types=[
            pltpu.VMEM((nch, CH), jnp.int32),
            pltpu.VMEM((CH,), jnp.float32),
            pltpu.VMEM((rpt,), jnp.float32),
            pltpu.VMEM_SHARED((n_pad,), jnp.float32),
        ],
    )
    def deg_kernel(dsts_hbm, out_hbm, idx_v, ones_v, z_v, deg_sh):
        cid = lax.axis_index("c")
        sid = lax.axis_index("s")
        wid = cid * NS + sid

        def initbody(i, _):
            ones_v[pl.ds(i * L, L)] = jnp.ones((L,), jnp.float32)
            return 0

        lax.fori_loop(0, CH // L, initbody, 0)

        def zbody(i, _):
            z_v[pl.ds(i * L, L)] = jnp.zeros((L,), jnp.float32)
            return 0

        lax.fori_loop(0, rpt // L, zbody, 0)
        pltpu.sync_copy(z_v, deg_sh.at[pl.ds(sid * rpt, rpt)])
        plsc.subcore_barrier()

        pltpu.sync_copy(dsts_hbm.at[wid], idx_v)

        def body(j, _):
            pltpu.sync_copy(ones_v, deg_sh.at[idx_v.at[j]], add=True)
            return 0

        lax.fori_loop(0, nch, body, 0)
        plsc.subcore_barrier()
        pltpu.sync_copy(deg_sh.at[pl.ds(sid * rpt, rpt)],
                        out_hbm.at[cid, pl.ds(sid * rpt, rpt)])

    return deg_kernel


def _make_prop_kernel(n_pad, f, nch):
    mesh = plsc.VectorSubcoreMesh(core_axis_name="c", subcore_axis_name="s")
    rpt = n_pad // NS
    nzc = rpt // CH
    npairs = nch // 2

    @functools.partial(
        pl.kernel,
        out_type=jax.ShapeDtypeStruct((NC, n_pad, f), jnp.float32),
        mesh=mesh,
        compiler_params=pltpu.CompilerParams(use_tc_tiling_on_sc=False),
        scratch_types=[
            pltpu.VMEM((nch, CH), jnp.int32),
            pltpu.VMEM((nch, CH), jnp.int32),
            pltpu.VMEM((CH, f), jnp.float32),
            pltpu.VMEM((CH, f), jnp.float32),
            pltpu.VMEM_SHARED((n_pad, f), jnp.float32),
            pltpu.SemaphoreType.DMA,
            pltpu.SemaphoreType.DMA,
        ],
    )
    def prop_kernel(feat_hbm, srcs_hbm, dsts_hbm, out_hbm,
                    si, di, b0, b1, acc_sh, s0, s1):
        cid = lax.axis_index("c")
        sid = lax.axis_index("s")
        wid = cid * NS + sid

        def zb(i, _):
            b0[i // (f // L), pl.ds((i % (f // L)) * L, L)] = (
                jnp.zeros((L,), jnp.float32))
            return 0

        lax.fori_loop(0, CH * (f // L), zb, 0)

        def zs(i, _):
            pltpu.sync_copy(b0, acc_sh.at[pl.ds(sid * rpt + i * CH, CH)])
            return 0

        lax.fori_loop(0, nzc, zs, 0)

        pltpu.sync_copy(srcs_hbm.at[wid], si)
        pltpu.sync_copy(dsts_hbm.at[wid], di)
        plsc.subcore_barrier()

        def fire(j, buf, sem):
            pltpu.async_copy(feat_hbm.at[si.at[j]], buf, sem)

        def wait(buf, sem):
            pltpu.make_async_copy(feat_hbm.at[pl.ds(0, CH)], buf, sem).wait()

        def scat(j, buf):
            pltpu.sync_copy(buf, acc_sh.at[di.at[j]], add=True)

        fire(0, b0, s0)

        def body(c, _):
            fire(2 * c + 1, b1, s1)
            wait(b0, s0)
            scat(2 * c, b0)
            fire(jnp.minimum(2 * c + 2, nch - 1), b0, s0)
            wait(b1, s1)
            scat(2 * c + 1, b1)
            return 0

        lax.fori_loop(0, npairs, body, 0)
        wait(b0, s0)
        if nch % 2 == 1:
            scat(nch - 1, b0)
        plsc.subcore_barrier()

        def wo(i, _):
            pltpu.sync_copy(acc_sh.at[pl.ds(sid * rpt + i * CH, CH)],
                            out_hbm.at[cid, pl.ds(sid * rpt + i * CH, CH)])
            return 0

        lax.fori_loop(0, nzc, wo, 0)

    return prop_kernel


def _scale_body(deg_ref, x_ref, xl_ref, xh_ref, s_ref):
    total = deg_ref[:, 0:1] + deg_ref[:, 1:2] + 1.0
    s = lax.rsqrt(total)
    s_ref[...] = s
    hw = x_ref.shape[1] // 2
    xl_ref[...] = x_ref[:, :hw] * s
    xh_ref[...] = x_ref[:, hw:] * s


def _dense_body(accl_ref, acch_ref, xl_ref, xh_ref, s_ref, w0_ref, w1_ref,
                q_ref):
    s = s_ref[...]
    hw = w0_ref.shape[0] // 2
    zl = (accl_ref[0, :, :] + accl_ref[1, :, :] + xl_ref[...]) * s
    zh = (acch_ref[0, :, :] + acch_ref[1, :, :] + xh_ref[...]) * s
    h = jnp.maximum(
        jnp.dot(zl, w0_ref[:hw, :], preferred_element_type=jnp.float32)
        + jnp.dot(zh, w0_ref[hw:, :], preferred_element_type=jnp.float32),
        0.0)
    q_ref[...] = jnp.dot(h * s, w1_ref[...],
                         preferred_element_type=jnp.float32)


def _final_body(acc_ref, q_ref, s_ref, o_ref):
    o_ref[...] = (acc_ref[0, :, :] + acc_ref[1, :, :] + q_ref[...]) * s_ref[...]


def kernel(x, edge_index, W0, W1):
    f32 = jnp.float32
    n, d = x.shape
    e = edge_index.shape[1]
    h = W0.shape[1]
    c = W1.shape[1]
    f1 = _round_up(c, L)
    n_pad = _round_up(n + 1, NS * CH)
    m_pad = _round_up(2 * e, NC * NS * CH)
    nch = m_pad // (NC * NS * CH)

    fill = jnp.full((m_pad - 2 * e,), n, jnp.int32)
    srcs = jnp.concatenate([edge_index[0], edge_index[1], fill])
    dsts = jnp.concatenate([edge_index[1], edge_index[0], fill])
    srcs = srcs.reshape(NC * NS, nch, CH)
    dsts = dsts.reshape(NC * NS, nch, CH)
    x_pad = jnp.zeros((n_pad, d), f32).at[:n].set(x)
    w1p = jnp.zeros((h, f1), f32).at[:, :c].set(W1)

    degp = _make_deg_kernel(n_pad, nch)(dsts)
    degt = degp.T

    hw = d // 2
    rb = 512
    grid = (n_pad // rb,)
    xs_l, xs_h, s2 = pl.pallas_call(
        _scale_body,
        grid=grid,
        in_specs=[
            pl.BlockSpec((rb, NC), lambda i: (i, 0)),
            pl.BlockSpec((rb, d), lambda i: (i, 0)),
        ],
        out_specs=[
            pl.BlockSpec((rb, hw), lambda i: (i, 0)),
            pl.BlockSpec((rb, hw), lambda i: (i, 0)),
            pl.BlockSpec((rb, 1), lambda i: (i, 0)),
        ],
        out_shape=[
            jax.ShapeDtypeStruct((n_pad, hw), f32),
            jax.ShapeDtypeStruct((n_pad, hw), f32),
            jax.ShapeDtypeStruct((n_pad, 1), f32),
        ],
    )(degt, x_pad)

    prop_hw = _make_prop_kernel(n_pad, hw, nch)
    accp_l = prop_hw(xs_l, srcs, dsts)
    accp_h = prop_hw(xs_h, srcs, dsts)

    q = pl.pallas_call(
        _dense_body,
        grid=grid,
        in_specs=[
            pl.BlockSpec((NC, rb, hw), lambda i: (0, i, 0)),
            pl.BlockSpec((NC, rb, hw), lambda i: (0, i, 0)),
            pl.BlockSpec((rb, hw), lambda i: (i, 0)),
            pl.BlockSpec((rb, hw), lambda i: (i, 0)),
            pl.BlockSpec((rb, 1), lambda i: (i, 0)),
            pl.BlockSpec((d, h), lambda i: (0, 0)),
            pl.BlockSpec((h, f1), lambda i: (0, 0)),
        ],
        out_specs=pl.BlockSpec((rb, f1), lambda i: (i, 0)),
        out_shape=jax.ShapeDtypeStruct((n_pad, f1), f32),
    )(accp_l, accp_h, xs_l, xs_h, s2, W0, w1p)

    accp2 = _make_prop_kernel(n_pad, f1, nch)(q, srcs, dsts)

    outp = pl.pallas_call(
        _final_body,
        grid=grid,
        in_specs=[
            pl.BlockSpec((NC, rb, f1), lambda i: (0, i, 0)),
            pl.BlockSpec((rb, f1), lambda i: (i, 0)),
            pl.BlockSpec((rb, 1), lambda i: (i, 0)),
        ],
        out_specs=pl.BlockSpec((rb, f1), lambda i: (i, 0)),
        out_shape=jax.ShapeDtypeStruct((n_pad, f1), f32),
    )(accp2, q, s2)

    return outp[:n, :c]
